# Initial kernel scaffold; baseline (speedup 1.0000x reference)
import jax
import jax.numpy as jnp
from jax.experimental import pallas as pl


def kernel(x, edge_index, adj_values):
    raise NotImplementedError("write your pallas kernel here")


# speedup vs baseline: 2.6070x; 2.6070x over previous
import dataclasses
import functools

import jax
import jax.numpy as jnp
from jax import lax
from jax.experimental import pallas as pl
from jax.experimental.pallas import tpu as pltpu
from jax.experimental.pallas import tpu_sc as plsc

N_NODES = 10000
N_EDGES = 160000
D_FEAT = 256
DH = 128
NC = 2
NS = 16
C = 80
EDGES_PER_SUB = N_EDGES // NS
NITER = EDGES_PER_SUB // C
N_PAD = 10240
ROWS_PER_SUB = N_PAD // NS
ZR = 128


def _gcn_sc_body(x2_hbm, src_hbm, dst_hbm, val_hbm, out_hbm,
                 src_v, dst_v, val_v, rows_v, zero_v, acc_sh):
    c = lax.axis_index("c")
    s = lax.axis_index("s")

    @pl.loop(0, ZR)
    def _(r):
        for k in range(DH // 16):
            zero_v.at[r, pl.ds(k * 16, 16)][...] = jnp.zeros((16,), jnp.float32)

    @pl.loop(0, ROWS_PER_SUB // ZR)
    def _(i):
        pltpu.sync_copy(zero_v, acc_sh.at[pl.ds(s * ROWS_PER_SUB + i * ZR, ZR)])

    plsc.subcore_barrier()

    ebase = s * EDGES_PER_SUB
    row_off16 = jnp.full((16,), c * N_NODES, jnp.int32)

    @pl.loop(0, NITER)
    def _(j):
        b = ebase + j * C
        pltpu.sync_copy(src_hbm.at[pl.ds(b, C)], src_v)
        pltpu.sync_copy(dst_hbm.at[pl.ds(b, C)], dst_v)
        pltpu.sync_copy(val_hbm.at[pl.ds(b, C)], val_v)

        @pl.loop(0, C // 16)
        def _(t):
            sl = src_v.at[pl.ds(t * 16, 16)]
            sl[...] = sl[...] + row_off16

        pltpu.sync_copy(x2_hbm.at[src_v], rows_v)

        @pl.loop(0, C)
        def _(e):
            e16 = jnp.full((16,), e, jnp.int32)
            v16 = plsc.load_gather(val_v, [e16])
            for k in range(DH // 16):
                sl = rows_v.at[e, pl.ds(k * 16, 16)]
                sl[...] = sl[...] * v16

        pltpu.sync_copy(rows_v, acc_sh.at[dst_v], add=True)

    plsc.subcore_barrier()

    @pl.loop(0, ROWS_PER_SUB // ZR)
    def _(i):
        r0 = s * ROWS_PER_SUB + i * ZR
        pltpu.sync_copy(acc_sh.at[pl.ds(r0, ZR)], out_hbm.at[c, pl.ds(r0, ZR)])


@jax.jit
def _gcn_sc(x2, src, dst, vals):
    mesh = plsc.VectorSubcoreMesh(core_axis_name="c", subcore_axis_name="s")
    cp = pltpu.CompilerParams()
    if "needs_layout_passes" in pltpu.CompilerParams.__dataclass_fields__:
        cp = dataclasses.replace(cp, needs_layout_passes=False)
    kern = functools.partial(
        pl.kernel,
        mesh=mesh,
        compiler_params=cp,
        out_type=jax.ShapeDtypeStruct((NC, N_PAD, DH), jnp.float32),
        scratch_types=[
            pltpu.VMEM((C,), jnp.int32),
            pltpu.VMEM((C,), jnp.int32),
            pltpu.VMEM((C,), jnp.float32),
            pltpu.VMEM((C, DH), jnp.float32),
            pltpu.VMEM((ZR, DH), jnp.float32),
            pltpu.VMEM_SHARED((N_PAD, DH), jnp.float32),
        ],
    )(_gcn_sc_body)
    return kern(x2, src, dst, vals)


def kernel(x, edge_index, adj_values):
    src = edge_index[0].astype(jnp.int32)
    dst = edge_index[1].astype(jnp.int32)
    vals = adj_values.astype(jnp.float32)
    x2 = jnp.concatenate([x[:, :DH], x[:, DH:]], axis=0)
    out2 = _gcn_sc(x2, src, dst, vals)
    return jnp.concatenate([out2[0, :N_NODES], out2[1, :N_NODES]], axis=1)


# speedup vs baseline: 5.8442x; 2.2417x over previous
import dataclasses
import functools

import jax
import jax.numpy as jnp
from jax import lax
from jax.experimental import pallas as pl
from jax.experimental.pallas import tpu as pltpu
from jax.experimental.pallas import tpu_sc as plsc

N_NODES = 10000
N_EDGES = 160000
D_FEAT = 256
DH = 128
NC = 2
NS = 16
C = 50
EDGES_PER_SUB = N_EDGES // NS
NITER = EDGES_PER_SUB // C
BLK = 8
NBLK = NITER // BLK
NB = 4
NI = 3
N_PAD = 10240
ROWS_PER_SUB = N_PAD // NS
ZR = 64


def _gcn_sc_body(x2_hbm, srcb_hbm, dst_hbm, val_hbm, out_hbm,
                 sv, dv, vv, b0, b1, b2, b3, zero_v, acc_sh,
                 sem_si, sem_di, sem_vi, sem_g, sem_s):
    c = lax.axis_index("c")
    s = lax.axis_index("s")
    bufs = (b0, b1, b2, b3)

    @pl.loop(0, ZR)
    def _(r):
        for k in range(DH // 16):
            zero_v.at[r, pl.ds(k * 16, 16)][...] = jnp.zeros((16,), jnp.float32)

    @pl.loop(0, ROWS_PER_SUB // ZR)
    def _(i):
        pltpu.sync_copy(zero_v, acc_sh.at[pl.ds(s * ROWS_PER_SUB + i * ZR, ZR)])

    plsc.subcore_barrier()

    def idx_descr(g, slot):
        j0 = pl.multiple_of(g * BLK, BLK)
        return (
            pltpu.make_async_copy(
                srcb_hbm.at[c, s, pl.ds(j0, BLK)], sv.at[slot], sem_si.at[slot]),
            pltpu.make_async_copy(
                dst_hbm.at[s, pl.ds(j0, BLK)], dv.at[slot], sem_di.at[slot]),
            pltpu.make_async_copy(
                val_hbm.at[s, pl.ds(j0, BLK)], vv.at[slot], sem_vi.at[slot]),
        )

    def gather_descr(slot, b, q):
        return pltpu.make_async_copy(
            x2_hbm.at[sv.at[slot, b]], bufs[q], sem_g.at[q])

    def scatter_descr(slot, b, q):
        return pltpu.make_async_copy(
            bufs[q], acc_sh.at[dv.at[slot, b]], sem_s.at[q])

    def scale_chunk(slot, b, q):
        p16 = jnp.full((16,), slot, jnp.int32)
        b16 = jnp.full((16,), b, jnp.int32)

        @pl.loop(0, C)
        def _(e):
            e16 = jnp.full((16,), e, jnp.int32)
            v16 = plsc.load_gather(vv, [p16, b16, e16])
            for k in range(DH // 16):
                sl = pl.ds(k * 16, 16)
                bufs[q].at[e, sl][...] = bufs[q].at[e, sl][...] * v16

    def do_block(g, slot, nslot, first, last):
        if not last:
            for d in idx_descr(g + 1, nslot):
                d.start()
        for b in range(BLK):
            q = b % NB
            qn = (b + 1) % NB
            if not (first and b < NB - 1):
                scatter_descr(slot, b, qn).wait()
            if b == BLK - 1:
                if not last:
                    for d in idx_descr(g + 1, nslot):
                        d.wait()
                    gather_descr(nslot, 0, qn).start()
            else:
                gather_descr(slot, b + 1, qn).start()
            gather_descr(slot, b, q).wait()
            scale_chunk(slot, b, q)
            pltpu.async_copy(
                bufs[q], acc_sh.at[dv.at[slot, b]], sem_s.at[q], add=True)

    for d in idx_descr(0, 0):
        d.start()
        d.wait()
    gather_descr(0, 0, 0).start()

    do_block(0, 0, 1, first=True, last=False)

    @pl.loop(1, NBLK - 3, step=NI)
    def _(g):
        do_block(g, 1, 2, first=False, last=False)
        do_block(g + 1, 2, 0, first=False, last=False)
        do_block(g + 2, 0, 1, first=False, last=False)

    do_block(NBLK - 3, 1, 2, first=False, last=False)
    do_block(NBLK - 2, 2, 0, first=False, last=False)
    do_block(NBLK - 1, 0, 1, first=False, last=True)

    for b in range(BLK - NB + 1, BLK):
        scatter_descr(0, b, b % NB).wait()

    plsc.subcore_barrier()

    @pl.loop(0, ROWS_PER_SUB // ZR)
    def _(i):
        r0 = s * ROWS_PER_SUB + i * ZR
        pltpu.sync_copy(acc_sh.at[pl.ds(r0, ZR)], out_hbm.at[c, pl.ds(r0, ZR)])


@jax.jit
def _gcn_sc(x2, srcb, dst2, val2):
    mesh = plsc.VectorSubcoreMesh(core_axis_name="c", subcore_axis_name="s")
    cp = pltpu.CompilerParams()
    if "needs_layout_passes" in pltpu.CompilerParams.__dataclass_fields__:
        cp = dataclasses.replace(cp, needs_layout_passes=False)
    kern = functools.partial(
        pl.kernel,
        mesh=mesh,
        compiler_params=cp,
        out_type=jax.ShapeDtypeStruct((NC, N_PAD, DH), jnp.float32),
        scratch_types=[
            pltpu.VMEM((NI, BLK, C), jnp.int32),
            pltpu.VMEM((NI, BLK, C), jnp.int32),
            pltpu.VMEM((NI, BLK, C), jnp.float32),
            pltpu.VMEM((C, DH), jnp.float32),
            pltpu.VMEM((C, DH), jnp.float32),
            pltpu.VMEM((C, DH), jnp.float32),
            pltpu.VMEM((C, DH), jnp.float32),
            pltpu.VMEM((ZR, DH), jnp.float32),
            pltpu.VMEM_SHARED((N_PAD, DH), jnp.float32),
            pltpu.SemaphoreType.DMA((NI,)),
            pltpu.SemaphoreType.DMA((NI,)),
            pltpu.SemaphoreType.DMA((NI,)),
            pltpu.SemaphoreType.DMA((NB,)),
            pltpu.SemaphoreType.DMA((NB,)),
        ],
    )(_gcn_sc_body)
    return kern(x2, srcb, dst2, val2)


def kernel(x, edge_index, adj_values):
    src = edge_index[0].astype(jnp.int32)
    dst = edge_index[1].astype(jnp.int32)
    vals = adj_values.astype(jnp.float32)
    x2 = jnp.concatenate([x[:, :DH], x[:, DH:]], axis=0)
    srcb = jnp.stack([src, src + N_NODES]).reshape(NC, NS, NITER, C)
    dst2 = dst.reshape(NS, NITER, C)
    val2 = vals.reshape(NS, NITER, C)
    out2 = _gcn_sc(x2, srcb, dst2, val2)
    return jnp.concatenate([out2[0, :N_NODES], out2[1, :N_NODES]], axis=1)


# speedup vs baseline: 6.4784x; 1.1085x over previous
import dataclasses
import functools

import jax
import jax.numpy as jnp
from jax import lax
from jax.experimental import pallas as pl
from jax.experimental.pallas import tpu as pltpu
from jax.experimental.pallas import tpu_sc as plsc

N_NODES = 10000
N_EDGES = 160000
D_FEAT = 256
DH = 128
NC = 2
NS = 16
C = 50
EDGES_PER_SUB = N_EDGES // NS
NITER = EDGES_PER_SUB // C
BLK = 8
NBLK = NITER // BLK
NB = 4
NI = 3
N_PAD = 10240
ROWS_PER_SUB = N_PAD // NS
ZR = 64


def _gcn_sc_body(x2_hbm, srcb_hbm, dst_hbm, val_hbm, out_hbm,
                 sv, dv, vv, b0, b1, b2, b3, zero_v, acc_sh,
                 sem_si, sem_di, sem_vi, sem_g, sem_s):
    c = lax.axis_index("c")
    s = lax.axis_index("s")
    bufs = (b0, b1, b2, b3)

    @pl.loop(0, ZR)
    def _(r):
        for k in range(DH // 16):
            zero_v.at[r, pl.ds(k * 16, 16)][...] = jnp.zeros((16,), jnp.float32)

    @pl.loop(0, ROWS_PER_SUB // ZR)
    def _(i):
        pltpu.sync_copy(zero_v, acc_sh.at[pl.ds(s * ROWS_PER_SUB + i * ZR, ZR)])

    plsc.subcore_barrier()

    def idx_descr(g, slot):
        j0 = pl.multiple_of(g * BLK, BLK)
        return (
            pltpu.make_async_copy(
                srcb_hbm.at[c, s, pl.ds(j0, BLK)], sv.at[slot], sem_si.at[slot]),
            pltpu.make_async_copy(
                dst_hbm.at[s, pl.ds(j0, BLK)], dv.at[slot], sem_di.at[slot]),
            pltpu.make_async_copy(
                val_hbm.at[s, pl.ds(j0, BLK)], vv.at[slot], sem_vi.at[slot]),
        )

    def gather_descr(slot, b, q):
        return pltpu.make_async_copy(
            x2_hbm.at[sv.at[slot, b]], bufs[q], sem_g.at[q])

    def scatter_descr(slot, b, q):
        return pltpu.make_async_copy(
            bufs[q], acc_sh.at[dv.at[slot, b]], sem_s.at[q])

    def scale_chunk(slot, b, q):
        p16 = jnp.full((16,), slot, jnp.int32)
        b16 = jnp.full((16,), b, jnp.int32)

        @plsc.parallel_loop(0, C, unroll=1)
        def _(e):
            e16 = jnp.full((16,), e, jnp.int32)
            v16 = plsc.load_gather(vv, [p16, b16, e16])
            for k in range(DH // 16):
                sl = pl.ds(k * 16, 16)
                bufs[q].at[e, sl][...] = bufs[q].at[e, sl][...] * v16

    def do_block(g, slot, nslot, first, last):
        if not last:
            for d in idx_descr(g + 1, nslot):
                d.start()
        for b in range(BLK):
            q = b % NB
            qn = (b + 1) % NB
            if not (first and b < NB - 1):
                scatter_descr(slot, b, qn).wait()
            if b == BLK - 1:
                if not last:
                    for d in idx_descr(g + 1, nslot):
                        d.wait()
                    gather_descr(nslot, 0, qn).start()
            else:
                gather_descr(slot, b + 1, qn).start()
            gather_descr(slot, b, q).wait()
            scale_chunk(slot, b, q)
            pltpu.async_copy(
                bufs[q], acc_sh.at[dv.at[slot, b]], sem_s.at[q], add=True)

    for d in idx_descr(0, 0):
        d.start()
        d.wait()
    gather_descr(0, 0, 0).start()

    do_block(0, 0, 1, first=True, last=False)

    @pl.loop(1, NBLK - 3, step=NI)
    def _(g):
        do_block(g, 1, 2, first=False, last=False)
        do_block(g + 1, 2, 0, first=False, last=False)
        do_block(g + 2, 0, 1, first=False, last=False)

    do_block(NBLK - 3, 1, 2, first=False, last=False)
    do_block(NBLK - 2, 2, 0, first=False, last=False)
    do_block(NBLK - 1, 0, 1, first=False, last=True)

    for b in range(BLK - NB + 1, BLK):
        scatter_descr(0, b, b % NB).wait()

    plsc.subcore_barrier()

    @pl.loop(0, ROWS_PER_SUB // ZR)
    def _(i):
        r0 = s * ROWS_PER_SUB + i * ZR
        pltpu.sync_copy(acc_sh.at[pl.ds(r0, ZR)], out_hbm.at[c, pl.ds(r0, ZR)])


@jax.jit
def _gcn_sc(x2, srcb, dst2, val2):
    mesh = plsc.VectorSubcoreMesh(core_axis_name="c", subcore_axis_name="s")
    cp = pltpu.CompilerParams()
    if "needs_layout_passes" in pltpu.CompilerParams.__dataclass_fields__:
        cp = dataclasses.replace(cp, needs_layout_passes=False)
    kern = functools.partial(
        pl.kernel,
        mesh=mesh,
        compiler_params=cp,
        out_type=jax.ShapeDtypeStruct((NC, N_PAD, DH), jnp.float32),
        scratch_types=[
            pltpu.VMEM((NI, BLK, C), jnp.int32),
            pltpu.VMEM((NI, BLK, C), jnp.int32),
            pltpu.VMEM((NI, BLK, C), jnp.float32),
            pltpu.VMEM((C, DH), jnp.float32),
            pltpu.VMEM((C, DH), jnp.float32),
            pltpu.VMEM((C, DH), jnp.float32),
            pltpu.VMEM((C, DH), jnp.float32),
            pltpu.VMEM((ZR, DH), jnp.float32),
            pltpu.VMEM_SHARED((N_PAD, DH), jnp.float32),
            pltpu.SemaphoreType.DMA((NI,)),
            pltpu.SemaphoreType.DMA((NI,)),
            pltpu.SemaphoreType.DMA((NI,)),
            pltpu.SemaphoreType.DMA((NB,)),
            pltpu.SemaphoreType.DMA((NB,)),
        ],
    )(_gcn_sc_body)
    return kern(x2, srcb, dst2, val2)


def kernel(x, edge_index, adj_values):
    src = edge_index[0].astype(jnp.int32)
    dst = edge_index[1].astype(jnp.int32)
    vals = adj_values.astype(jnp.float32)
    x2 = jnp.concatenate([x[:, :DH], x[:, DH:]], axis=0)
    srcb = jnp.stack([src, src + N_NODES]).reshape(NC, NS, NITER, C)
    dst2 = dst.reshape(NS, NITER, C)
    val2 = vals.reshape(NS, NITER, C)
    out2 = _gcn_sc(x2, srcb, dst2, val2)
    return jnp.concatenate([out2[0, :N_NODES], out2[1, :N_NODES]], axis=1)
